# Initial kernel scaffold; baseline (speedup 1.0000x reference)
import jax
import jax.numpy as jnp
from jax.experimental import pallas as pl


def kernel(feats, coords, W, b):
    raise NotImplementedError("write your pallas kernel here")


# speedup vs baseline: 2.5926x; 2.5926x over previous
import functools

import jax
import jax.numpy as jnp
from jax import lax
from jax.experimental import pallas as pl
from jax.experimental.pallas import tpu as pltpu
from jax.experimental.pallas import tpu_sc as plsc

N = 100000
CIN = 32
COUT = 32
KK = 27
B_, D_, H_, W_ = 4, 64, 64, 64

NP = 102400
CH = 3200
SUB = 128
NSUB = CH // SUB
CHA = 6400
NSUBA = CHA // SUB

VOL = B_ * D_ * H_ * W_
VPAD = 4096
VOLP = VOL + VPAD
ZROWS = 512
NF = N + ZROWS

MEMW = VOLP // 16
MB = 8192
NMB = MEMW // MB
MREM = MEMW - NMB * MB

_mesh = plsc.VectorSubcoreMesh(core_axis_name="c", subcore_axis_name="s")
_sc_params = pltpu.CompilerParams(use_tc_tiling_on_sc=False)


def _iota16():
    return lax.iota(jnp.int32, 16)


def _build_vol(cbi, czi, cyi, cxi):
    @functools.partial(
        pl.kernel,
        out_type=jax.ShapeDtypeStruct((2 * VOLP,), jnp.int32),
        mesh=_mesh,
        compiler_params=_sc_params,
        scratch_types=[
            pltpu.VMEM((MB,), jnp.int32),
            pltpu.VMEM((CHA,), jnp.int32),
            pltpu.VMEM((CHA,), jnp.int32),
            pltpu.VMEM((CHA,), jnp.int32),
            pltpu.VMEM((CHA,), jnp.int32),
            pltpu.VMEM((NSUBA, SUB), jnp.int32),
            pltpu.VMEM((NSUBA, SUB), jnp.int32),
            pltpu.SemaphoreType.DMA,
        ],
    )
    def k(cb_h, cz_h, cy_h, cx_h, volf, mbuf, cb, cz, cy, cx, fl2, id2, sem):
        c = lax.axis_index("c")
        s = lax.axis_index("s")
        co = c * VOLP

        def fill(i, _):
            mbuf[pl.ds(i * 16, 16)] = N + ((i * 16 + _iota16()) & (ZROWS - 1))
            return 0

        lax.fori_loop(0, MB // 16, fill, 0)

        base_m = co + s * MEMW

        def mset(i, _):
            pltpu.sync_copy(mbuf, volf.at[pl.ds(base_m + i * MB, MB)])
            return 0

        lax.fori_loop(0, NMB, mset, 0)
        pltpu.sync_copy(mbuf.at[pl.ds(0, MREM)],
                        volf.at[pl.ds(base_m + NMB * MB, MREM)])

        plsc.subcore_barrier()

        pb = s * CHA
        pltpu.sync_copy(cb_h.at[pl.ds(pb, CHA)], cb)
        pltpu.sync_copy(cz_h.at[pl.ds(pb, CHA)], cz)
        pltpu.sync_copy(cy_h.at[pl.ds(pb, CHA)], cy)
        pltpu.sync_copy(cx_h.at[pl.ds(pb, CHA)], cx)

        def comp(g, _):
            off = g * 16
            r = g >> 3
            u = (g & 7) * 16
            gid = pb + off + _iota16()
            b16 = cb[pl.ds(off, 16)]
            z16 = cz[pl.ds(off, 16)]
            y16 = cy[pl.ds(off, 16)]
            x16 = cx[pl.ds(off, 16)]
            flat = ((b16 * D_ + z16) * H_ + y16) * W_ + x16
            ispad = gid >= N
            flat = jnp.where(ispad, VOL + (gid & (VPAD - 1)), flat)
            idv = jnp.where(ispad, N + (gid & (ZROWS - 1)), gid)
            fl2[r, pl.ds(u, 16)] = co + flat
            id2[r, pl.ds(u, 16)] = idv
            return 0

        lax.fori_loop(0, CHA // 16, comp, 0)

        def fire(j, _):
            pltpu.async_copy(id2.at[j], volf.at[fl2.at[j]], sem)
            return 0

        lax.fori_loop(0, NSUBA, fire, 0)

        def drain(j, _):
            pltpu.make_async_copy(id2.at[j], volf.at[fl2.at[j]], sem).wait()
            return 0

        lax.fori_loop(0, NSUBA, drain, 0)

    return k(cbi, czi, cyi, cxi)


def _rulebook_gather(volf, cbi, czi, cyi, cxi, fx):
    QL = 5
    QROWS = QL * SUB

    @functools.partial(
        pl.kernel,
        out_type=jax.ShapeDtypeStruct((KK * NP, CIN), jnp.float32),
        mesh=_mesh,
        compiler_params=_sc_params,
        scratch_types=[
            pltpu.VMEM((CH,), jnp.int32),
            pltpu.VMEM((CH,), jnp.int32),
            pltpu.VMEM((CH,), jnp.int32),
            pltpu.VMEM((CH,), jnp.int32),
            pltpu.VMEM((NSUB, SUB), jnp.int32),
            pltpu.VMEM((NSUB, SUB), jnp.int32),
            pltpu.VMEM((NSUB, SUB), jnp.int32),
            pltpu.VMEM((QROWS, CIN), jnp.float32),
            pltpu.SemaphoreType.DMA,
        ],
    )
    def k(volf_h, cb_h, cz_h, cy_h, cx_h, fx_h, g,
          cb, cz, cy, cx, fl2, nix, nid, rows, sem):
        c = lax.axis_index("c")
        s = lax.axis_index("s")
        wid = s * 2 + c
        co = c * VOLP
        pb = wid * CH

        pltpu.sync_copy(cb_h.at[pl.ds(pb, CH)], cb)
        pltpu.sync_copy(cz_h.at[pl.ds(pb, CH)], cz)
        pltpu.sync_copy(cy_h.at[pl.ds(pb, CH)], cy)
        pltpu.sync_copy(cx_h.at[pl.ds(pb, CH)], cx)

        def comp(g_, _):
            off = g_ * 16
            r = g_ >> 3
            u = (g_ & 7) * 16
            gid = pb + off + _iota16()
            b16 = cb[pl.ds(off, 16)]
            z16 = cz[pl.ds(off, 16)]
            y16 = cy[pl.ds(off, 16)]
            x16 = cx[pl.ds(off, 16)]
            flat = ((b16 * D_ + z16) * H_ + y16) * W_ + x16
            ispad = gid >= N
            flat = jnp.where(ispad, VOL + (gid & (VPAD - 1)), flat)
            fl2[r, pl.ds(u, 16)] = flat
            return 0

        lax.fori_loop(0, CH // 16, comp, 0)

        def per_k(kk, _):
            dz = kk // 9 - 1
            dy = (kk // 3) % 3 - 1
            dx = kk % 3 - 1
            delta = (dz * H_ + dy) * W_ + dx

            def comp2(g_, _):
                r = g_ >> 3
                u = (g_ & 7) * 16
                flat = fl2[r, pl.ds(u, 16)]
                live = flat < VOL
                z16 = (flat >> 12) & 63
                y16 = (flat >> 6) & 63
                x16 = flat & 63
                ok = (live
                      & (z16 + dz >= 0) & (z16 + dz < D_)
                      & (y16 + dy >= 0) & (y16 + dy < H_)
                      & (x16 + dx >= 0) & (x16 + dx < W_))
                a = jnp.where(ok, flat + delta, VOL + (flat & (VPAD - 1)))
                nix[r, pl.ds(u, 16)] = co + a
                return 0

            lax.fori_loop(0, CH // 16, comp2, 0)

            def fire1(j, _):
                pltpu.async_copy(volf_h.at[nix.at[j]], nid.at[j], sem)
                return 0

            lax.fori_loop(0, NSUB, fire1, 0)

            def drain1(j, _):
                pltpu.make_async_copy(volf_h.at[nix.at[j]], nid.at[j],
                                      sem).wait()
                return 0

            lax.fori_loop(0, NSUB, drain1, 0)

            def quint(q, _):
                def fire2(t, _):
                    j = q * QL + t
                    pltpu.async_copy(fx_h.at[nid.at[j]],
                                     rows.at[pl.ds(t * SUB, SUB), :], sem)
                    return 0

                lax.fori_loop(0, QL, fire2, 0)

                def drain2(t, _):
                    j = q * QL + t
                    pltpu.make_async_copy(fx_h.at[nid.at[j]],
                                          rows.at[pl.ds(t * SUB, SUB), :],
                                          sem).wait()
                    return 0

                lax.fori_loop(0, QL, drain2, 0)
                pltpu.sync_copy(
                    rows,
                    g.at[pl.ds(kk * NP + pb + q * QROWS, QROWS), :])
                return 0

            lax.fori_loop(0, NSUB // QL, quint, 0)
            return 0

        lax.fori_loop(0, KK, per_k, 0)

    return k(volf, cbi, czi, cyi, cxi, fx)


def _matmul(g, wf, bias):
    M = 4096

    def body(g_ref, w_ref, b_ref, o_ref):
        k = pl.program_id(1)
        d = jnp.dot(g_ref[...], w_ref[...],
                    preferred_element_type=jnp.float32)

        @pl.when(k == 0)
        def _():
            o_ref[...] = d + b_ref[...]

        @pl.when(k > 0)
        def _():
            o_ref[...] += d

    return pl.pallas_call(
        body,
        grid=(NP // M, KK),
        in_specs=[
            pl.BlockSpec((M, CIN), lambda i, k: (k * (NP // M) + i, 0)),
            pl.BlockSpec((CIN, COUT), lambda i, k: (k, 0)),
            pl.BlockSpec((1, COUT), lambda i, k: (0, 0)),
        ],
        out_specs=pl.BlockSpec((M, COUT), lambda i, k: (i, 0)),
        out_shape=jax.ShapeDtypeStruct((NP, COUT), jnp.float32),
    )(g, wf, bias)


def kernel(feats, coords, W, b):
    dtype_ = feats.dtype
    coords = coords.astype(jnp.int32)
    zpad = jnp.zeros((NP - N,), jnp.int32)
    cbi = jnp.concatenate([coords[:, 0], zpad])
    czi = jnp.concatenate([coords[:, 1], zpad])
    cyi = jnp.concatenate([coords[:, 2], zpad])
    cxi = jnp.concatenate([coords[:, 3], zpad])
    fx = jnp.concatenate(
        [feats.astype(jnp.float32),
         jnp.zeros((ZROWS, CIN), jnp.float32)], axis=0)
    volf = _build_vol(cbi, czi, cyi, cxi)
    g = _rulebook_gather(volf, cbi, czi, cyi, cxi, fx)
    out = _matmul(g,
                  W.reshape(KK * CIN, COUT).astype(jnp.float32),
                  b.reshape(1, COUT).astype(jnp.float32))
    return out[:N].astype(dtype_)


# speedup vs baseline: 4.0033x; 1.5441x over previous
import functools

import jax
import jax.numpy as jnp
from jax import lax
from jax.experimental import pallas as pl
from jax.experimental.pallas import tpu as pltpu
from jax.experimental.pallas import tpu_sc as plsc

N = 100000
CIN = 32
COUT = 32
KK = 27
KG = 7
B_, D_, H_, W_ = 4, 64, 64, 64

NP = 102400
CH = 3200
SUB = 128
NSUB = CH // SUB
CHA = 6400
NSUBA = CHA // SUB

VOL = B_ * D_ * H_ * W_
VPAD = 4096
VOLP = VOL + VPAD
ZROWS = 512
NF = N + ZROWS

MEMW = VOLP // 16
MB = 8192
NMB = MEMW // MB
MREM = MEMW - NMB * MB

_mesh = plsc.VectorSubcoreMesh(core_axis_name="c", subcore_axis_name="s")
_sc_params = pltpu.CompilerParams(use_tc_tiling_on_sc=False)


def _iota16():
    return lax.iota(jnp.int32, 16)


def _build_vol(cbi, czi, cyi, cxi):
    @functools.partial(
        pl.kernel,
        out_type=jax.ShapeDtypeStruct((2 * VOLP,), jnp.int32),
        mesh=_mesh,
        compiler_params=_sc_params,
        scratch_types=[
            pltpu.VMEM((MB,), jnp.int32),
            pltpu.VMEM((CHA,), jnp.int32),
            pltpu.VMEM((CHA,), jnp.int32),
            pltpu.VMEM((CHA,), jnp.int32),
            pltpu.VMEM((CHA,), jnp.int32),
            pltpu.VMEM((NSUBA, SUB), jnp.int32),
            pltpu.VMEM((NSUBA, SUB), jnp.int32),
            pltpu.SemaphoreType.DMA,
        ],
    )
    def k(cb_h, cz_h, cy_h, cx_h, volf, mbuf, cb, cz, cy, cx, fl2, id2, sem):
        c = lax.axis_index("c")
        s = lax.axis_index("s")
        co = c * VOLP

        def fill(i, _):
            mbuf[pl.ds(i * 16, 16)] = N + ((i * 16 + _iota16()) & (ZROWS - 1))
            return 0

        lax.fori_loop(0, MB // 16, fill, 0)

        base_m = co + s * MEMW

        def mset(i, _):
            pltpu.async_copy(mbuf, volf.at[pl.ds(base_m + i * MB, MB)], sem)
            return 0

        lax.fori_loop(0, NMB, mset, 0)
        pltpu.async_copy(mbuf.at[pl.ds(0, MREM)],
                         volf.at[pl.ds(base_m + NMB * MB, MREM)], sem)

        pb = s * CHA
        pltpu.sync_copy(cb_h.at[pl.ds(pb, CHA)], cb)
        pltpu.sync_copy(cz_h.at[pl.ds(pb, CHA)], cz)
        pltpu.sync_copy(cy_h.at[pl.ds(pb, CHA)], cy)
        pltpu.sync_copy(cx_h.at[pl.ds(pb, CHA)], cx)

        def comp(g, _):
            off = g * 16
            r = g >> 3
            u = (g & 7) * 16
            gid = pb + off + _iota16()
            b16 = cb[pl.ds(off, 16)]
            z16 = cz[pl.ds(off, 16)]
            y16 = cy[pl.ds(off, 16)]
            x16 = cx[pl.ds(off, 16)]
            flat = ((b16 * D_ + z16) * H_ + y16) * W_ + x16
            ispad = gid >= N
            flat = jnp.where(ispad, VOL + (gid & (VPAD - 1)), flat)
            idv = jnp.where(ispad, N + (gid & (ZROWS - 1)), gid)
            fl2[r, pl.ds(u, 16)] = co + flat
            id2[r, pl.ds(u, 16)] = idv
            return 0

        lax.fori_loop(0, CHA // 16, comp, 0)

        def mdrain(i, _):
            pltpu.make_async_copy(
                mbuf, volf.at[pl.ds(base_m + i * MB, MB)], sem).wait()
            return 0

        lax.fori_loop(0, NMB, mdrain, 0)
        pltpu.make_async_copy(
            mbuf.at[pl.ds(0, MREM)],
            volf.at[pl.ds(base_m + NMB * MB, MREM)], sem).wait()
        plsc.subcore_barrier()

        def fire(j, _):
            pltpu.async_copy(id2.at[j], volf.at[fl2.at[j]], sem)
            return 0

        lax.fori_loop(0, NSUBA, fire, 0)

        def drain(j, _):
            pltpu.make_async_copy(id2.at[j], volf.at[fl2.at[j]], sem).wait()
            return 0

        lax.fori_loop(0, NSUBA, drain, 0)

    return k(cbi, czi, cyi, cxi)


def _rulebook_gather(volf, cbi, czi, cyi, cxi, fx):
    QL = 5
    QROWS = QL * SUB

    @functools.partial(
        pl.kernel,
        out_type=jax.ShapeDtypeStruct((KG * NP, 128), jnp.bfloat16),
        mesh=_mesh,
        compiler_params=_sc_params,
        scratch_types=[
            pltpu.VMEM((CH,), jnp.int32),
            pltpu.VMEM((CH,), jnp.int32),
            pltpu.VMEM((CH,), jnp.int32),
            pltpu.VMEM((CH,), jnp.int32),
            pltpu.VMEM((NSUB, SUB), jnp.int32),
            pltpu.VMEM((NSUB, SUB), jnp.int32),
            pltpu.VMEM((NSUB, SUB), jnp.int32),
            pltpu.VMEM((QROWS, CIN), jnp.bfloat16),
            pltpu.SemaphoreType.DMA,
        ],
    )
    def k(volf_h, cb_h, cz_h, cy_h, cx_h, fx_h, g,
          cb, cz, cy, cx, fl2, nix, nid, rows, sem):
        c = lax.axis_index("c")
        s = lax.axis_index("s")
        wid = s * 2 + c
        co = c * VOLP
        pb = wid * CH

        pltpu.sync_copy(cb_h.at[pl.ds(pb, CH)], cb)
        pltpu.sync_copy(cz_h.at[pl.ds(pb, CH)], cz)
        pltpu.sync_copy(cy_h.at[pl.ds(pb, CH)], cy)
        pltpu.sync_copy(cx_h.at[pl.ds(pb, CH)], cx)

        def comp(g_, _):
            off = g_ * 16
            r = g_ >> 3
            u = (g_ & 7) * 16
            gid = pb + off + _iota16()
            b16 = cb[pl.ds(off, 16)]
            z16 = cz[pl.ds(off, 16)]
            y16 = cy[pl.ds(off, 16)]
            x16 = cx[pl.ds(off, 16)]
            flat = ((b16 * D_ + z16) * H_ + y16) * W_ + x16
            ispad = gid >= N
            flat = jnp.where(ispad, VOL + (gid & (VPAD - 1)), flat)
            fl2[r, pl.ds(u, 16)] = flat
            return 0

        lax.fori_loop(0, CH // 16, comp, 0)

        def per_k(kk, _):
            dz = jnp.where(kk < KK, kk // 9 - 1, 99)
            dy = (kk // 3) % 3 - 1
            dx = kk % 3 - 1
            delta = (dz * H_ + dy) * W_ + dx

            def comp2(g_, _):
                r = g_ >> 3
                u = (g_ & 7) * 16
                flat = fl2[r, pl.ds(u, 16)]
                live = flat < VOL
                z16 = (flat >> 12) & 63
                y16 = (flat >> 6) & 63
                x16 = flat & 63
                ok = (live
                      & (z16 + dz >= 0) & (z16 + dz < D_)
                      & (y16 + dy >= 0) & (y16 + dy < H_)
                      & (x16 + dx >= 0) & (x16 + dx < W_))
                a = jnp.where(ok, flat + delta, VOL + (flat & (VPAD - 1)))
                nix[r, pl.ds(u, 16)] = co + a
                return 0

            lax.fori_loop(0, CH // 16, comp2, 0)

            def fire1(j, _):
                pltpu.async_copy(volf_h.at[nix.at[j]], nid.at[j], sem)
                return 0

            lax.fori_loop(0, NSUB, fire1, 0)

            def drain1(j, _):
                pltpu.make_async_copy(volf_h.at[nix.at[j]], nid.at[j],
                                      sem).wait()
                return 0

            lax.fori_loop(0, NSUB, drain1, 0)

            def quint(q, _):
                def fire2(t, _):
                    j = q * QL + t
                    pltpu.async_copy(fx_h.at[nid.at[j]],
                                     rows.at[pl.ds(t * SUB, SUB), :], sem)
                    return 0

                lax.fori_loop(0, QL, fire2, 0)

                def drain2(t, _):
                    j = q * QL + t
                    pltpu.make_async_copy(fx_h.at[nid.at[j]],
                                          rows.at[pl.ds(t * SUB, SUB), :],
                                          sem).wait()
                    return 0

                lax.fori_loop(0, QL, drain2, 0)
                qg = kk >> 2
                jj = kk & 3
                pltpu.sync_copy(
                    rows,
                    g.at[pl.ds(qg * NP + pb + q * QROWS, QROWS),
                         pl.ds(jj * CIN, CIN)])
                return 0

            lax.fori_loop(0, NSUB // QL, quint, 0)
            return 0

        lax.fori_loop(0, 4 * KG, per_k, 0)

    return k(volf, cbi, czi, cyi, cxi, fx)


def _matmul(g, wf, bias):
    M = 4096

    def body(g_ref, w_ref, b_ref, o_ref):
        k = pl.program_id(1)
        d = jnp.dot(g_ref[...], w_ref[...],
                    preferred_element_type=jnp.float32)

        @pl.when(k == 0)
        def _():
            o_ref[...] = d + b_ref[...]

        @pl.when(k > 0)
        def _():
            o_ref[...] += d

    return pl.pallas_call(
        body,
        grid=(NP // M, KG),
        in_specs=[
            pl.BlockSpec((M, 128), lambda i, k: (k * (NP // M) + i, 0)),
            pl.BlockSpec((128, COUT), lambda i, k: (k, 0)),
            pl.BlockSpec((1, COUT), lambda i, k: (0, 0)),
        ],
        out_specs=pl.BlockSpec((M, COUT), lambda i, k: (i, 0)),
        out_shape=jax.ShapeDtypeStruct((NP, COUT), jnp.float32),
    )(g, wf, bias)


def kernel(feats, coords, W, b):
    dtype_ = feats.dtype
    coords = coords.astype(jnp.int32)
    zpad = jnp.zeros((NP - N,), jnp.int32)
    cbi = jnp.concatenate([coords[:, 0], zpad])
    czi = jnp.concatenate([coords[:, 1], zpad])
    cyi = jnp.concatenate([coords[:, 2], zpad])
    cxi = jnp.concatenate([coords[:, 3], zpad])
    fx = jnp.concatenate(
        [feats.astype(jnp.bfloat16),
         jnp.zeros((ZROWS, CIN), jnp.bfloat16)], axis=0)
    volf = _build_vol(cbi, czi, cyi, cxi)
    g = _rulebook_gather(volf, cbi, czi, cyi, cxi, fx)
    wpad = jnp.concatenate(
        [W.astype(jnp.float32),
         jnp.zeros((4 * KG - KK, CIN, COUT), jnp.float32)], axis=0)
    w4 = wpad.reshape(KG * 4 * CIN, COUT).astype(jnp.bfloat16)
    out = _matmul(g, w4, b.reshape(1, COUT).astype(jnp.float32))
    return out[:N].astype(dtype_)


# speedup vs baseline: 4.0385x; 1.0088x over previous
import functools

import jax
import jax.numpy as jnp
from jax import lax
from jax.experimental import pallas as pl
from jax.experimental.pallas import tpu as pltpu
from jax.experimental.pallas import tpu_sc as plsc

N = 100000
CIN = 32
COUT = 32
KK = 27
KG = 7
B_, D_, H_, W_ = 4, 64, 64, 64
DP = 66

NP = 102400
CH = 3200
SUB = 128
NSUB = CH // SUB
CHA = 6400
NSUBA = CHA // SUB
NPB = NP // SUB

VOLB = B_ * DP * DP * DP
SAFE = (DP + 1) * DP + 1
VPADR = 4096
VOLP = 1163008
ZROWS = 512
NF = N + ZROWS

MEMW = VOLP // 16
MB = 8192
NMB = MEMW // MB
MREM = MEMW - NMB * MB

_mesh = plsc.VectorSubcoreMesh(core_axis_name="c", subcore_axis_name="s")
_sc_params = pltpu.CompilerParams(use_tc_tiling_on_sc=False)


def _iota16():
    return lax.iota(jnp.int32, 16)


def _build_vol(cbi, czi, cyi, cxi):
    @functools.partial(
        pl.kernel,
        out_type=(jax.ShapeDtypeStruct((2 * VOLP,), jnp.int32),
                  jax.ShapeDtypeStruct((NPB, SUB), jnp.int32)),
        mesh=_mesh,
        compiler_params=_sc_params,
        scratch_types=[
            pltpu.VMEM((MB,), jnp.int32),
            pltpu.VMEM((CHA,), jnp.int32),
            pltpu.VMEM((CHA,), jnp.int32),
            pltpu.VMEM((CHA,), jnp.int32),
            pltpu.VMEM((CHA,), jnp.int32),
            pltpu.VMEM((NSUBA, SUB), jnp.int32),
            pltpu.VMEM((NSUBA, SUB), jnp.int32),
            pltpu.VMEM((NSUBA, SUB), jnp.int32),
            pltpu.SemaphoreType.DMA,
        ],
    )
    def k(cb_h, cz_h, cy_h, cx_h, volf, flat3,
          mbuf, cb, cz, cy, cx, fl2, flc, id2, sem):
        c = lax.axis_index("c")
        s = lax.axis_index("s")
        co = c * VOLP

        def fill(i, _):
            mbuf[pl.ds(i * 16, 16)] = N + ((i * 16 + _iota16()) & (ZROWS - 1))
            return 0

        lax.fori_loop(0, MB // 16, fill, 0)

        base_m = co + s * MEMW

        def mset(i, _):
            pltpu.async_copy(mbuf, volf.at[pl.ds(base_m + i * MB, MB)], sem)
            return 0

        lax.fori_loop(0, NMB, mset, 0)
        pltpu.async_copy(mbuf.at[pl.ds(0, MREM)],
                         volf.at[pl.ds(base_m + NMB * MB, MREM)], sem)

        pb = s * CHA
        pltpu.sync_copy(cb_h.at[pl.ds(pb, CHA)], cb)
        pltpu.sync_copy(cz_h.at[pl.ds(pb, CHA)], cz)
        pltpu.sync_copy(cy_h.at[pl.ds(pb, CHA)], cy)
        pltpu.sync_copy(cx_h.at[pl.ds(pb, CHA)], cx)

        def comp(g, _):
            off = g * 16
            r = g >> 3
            u = (g & 7) * 16
            gid = pb + off + _iota16()
            b16 = cb[pl.ds(off, 16)]
            z16 = cz[pl.ds(off, 16)]
            y16 = cy[pl.ds(off, 16)]
            x16 = cx[pl.ds(off, 16)]
            flat = ((b16 * DP + z16 + 1) * DP + y16 + 1) * DP + x16 + 1
            ispad = gid >= N
            flat = jnp.where(ispad, VOLB + SAFE + (gid & (VPADR - 1)), flat)
            idv = jnp.where(ispad, N + (gid & (ZROWS - 1)), gid)
            fl2[r, pl.ds(u, 16)] = flat
            flc[r, pl.ds(u, 16)] = co + flat
            id2[r, pl.ds(u, 16)] = idv
            return 0

        lax.fori_loop(0, CHA // 16, comp, 0)
        pltpu.sync_copy(fl2, flat3.at[pl.ds(s * NSUBA, NSUBA), :])

        def mdrain(i, _):
            pltpu.make_async_copy(
                mbuf, volf.at[pl.ds(base_m + i * MB, MB)], sem).wait()
            return 0

        lax.fori_loop(0, NMB, mdrain, 0)
        pltpu.make_async_copy(
            mbuf.at[pl.ds(0, MREM)],
            volf.at[pl.ds(base_m + NMB * MB, MREM)], sem).wait()
        plsc.subcore_barrier()

        def fire(j, _):
            pltpu.async_copy(id2.at[j], volf.at[flc.at[j]], sem)
            return 0

        lax.fori_loop(0, NSUBA, fire, 0)

        def drain(j, _):
            pltpu.make_async_copy(id2.at[j], volf.at[flc.at[j]], sem).wait()
            return 0

        lax.fori_loop(0, NSUBA, drain, 0)

    return k(cbi, czi, cyi, cxi)


def _rulebook_gather(volf, flat3, fx):
    @functools.partial(
        pl.kernel,
        out_type=jax.ShapeDtypeStruct((KG * NPB, SUB, 128), jnp.bfloat16),
        mesh=_mesh,
        compiler_params=_sc_params,
        scratch_types=[
            pltpu.VMEM((NSUB, SUB), jnp.int32),
            pltpu.VMEM((NSUB, SUB), jnp.int32),
            pltpu.VMEM((NSUB, SUB), jnp.int32),
            pltpu.VMEM((NSUB, SUB, CIN), jnp.bfloat16),
            pltpu.SemaphoreType.DMA,
        ],
    )
    def k(volf_h, flat3_h, fx_h, g, fl2, nix, nid, rows, sem):
        c = lax.axis_index("c")
        s = lax.axis_index("s")
        wid = s * 2 + c
        co = c * VOLP

        pltpu.sync_copy(flat3_h.at[pl.ds(wid * NSUB, NSUB), :], fl2)

        def per_k(kk, _):
            dz = kk // 9 - 1
            dy = (kk // 3) % 3 - 1
            dx = kk % 3 - 1
            delta = jnp.where(kk < KK, (dz * DP + dy) * DP + dx, 0)
            addc = co + delta

            def comp2(g_, _):
                r = g_ >> 3
                u = (g_ & 7) * 16
                nix[r, pl.ds(u, 16)] = fl2[r, pl.ds(u, 16)] + addc
                return 0

            lax.fori_loop(0, CH // 16, comp2, 0)

            def fire1(j, _):
                pltpu.async_copy(volf_h.at[nix.at[j]], nid.at[j], sem)
                return 0

            lax.fori_loop(0, NSUB, fire1, 0)

            def drain1(j, _):
                pltpu.make_async_copy(volf_h.at[nix.at[j]], nid.at[j],
                                      sem).wait()
                return 0

            lax.fori_loop(0, NSUB, drain1, 0)

            def fire2(j, _):
                pltpu.async_copy(fx_h.at[nid.at[j]], rows.at[j], sem)
                return 0

            lax.fori_loop(0, NSUB, fire2, 0)

            def drain2(j, _):
                pltpu.make_async_copy(fx_h.at[nid.at[j]], rows.at[j],
                                      sem).wait()
                return 0

            lax.fori_loop(0, NSUB, drain2, 0)

            qg = kk >> 2
            jj = kk & 3
            pltpu.sync_copy(
                rows,
                g.at[pl.ds(qg * NPB + wid * NSUB, NSUB), :,
                     pl.ds(jj * CIN, CIN)])
            return 0

        lax.fori_loop(0, 4 * KG, per_k, 0)

    return k(volf, flat3, fx)


def _matmul(g, wf, bias):
    M = 4096

    def body(g_ref, w_ref, b_ref, o_ref):
        k = pl.program_id(1)
        d = jnp.dot(g_ref[...], w_ref[...],
                    preferred_element_type=jnp.float32)

        @pl.when(k == 0)
        def _():
            o_ref[...] = d + b_ref[...]

        @pl.when(k > 0)
        def _():
            o_ref[...] += d

    return pl.pallas_call(
        body,
        grid=(NP // M, KG),
        in_specs=[
            pl.BlockSpec((M, 128), lambda i, k: (k * (NP // M) + i, 0)),
            pl.BlockSpec((128, COUT), lambda i, k: (k, 0)),
            pl.BlockSpec((1, COUT), lambda i, k: (0, 0)),
        ],
        out_specs=pl.BlockSpec((M, COUT), lambda i, k: (i, 0)),
        out_shape=jax.ShapeDtypeStruct((NP, COUT), jnp.float32),
    )(g, wf, bias)


def kernel(feats, coords, W, b):
    dtype_ = feats.dtype
    coords = coords.astype(jnp.int32)
    zpad = jnp.zeros((NP - N,), jnp.int32)
    cbi = jnp.concatenate([coords[:, 0], zpad])
    czi = jnp.concatenate([coords[:, 1], zpad])
    cyi = jnp.concatenate([coords[:, 2], zpad])
    cxi = jnp.concatenate([coords[:, 3], zpad])
    fx = jnp.concatenate(
        [feats.astype(jnp.bfloat16),
         jnp.zeros((ZROWS, CIN), jnp.bfloat16)], axis=0)
    volf, flat3 = _build_vol(cbi, czi, cyi, cxi)
    g3 = _rulebook_gather(volf, flat3, fx)
    g = g3.reshape(KG * NP, 128)
    wpad = jnp.concatenate(
        [W.astype(jnp.float32),
         jnp.zeros((4 * KG - KK, CIN, COUT), jnp.float32)], axis=0)
    w4 = wpad.reshape(KG * 4 * CIN, COUT).astype(jnp.bfloat16)
    out = _matmul(g, w4, b.reshape(1, COUT).astype(jnp.float32))
    return out[:N].astype(dtype_)


# speedup vs baseline: 4.0619x; 1.0058x over previous
import functools

import jax
import jax.numpy as jnp
from jax import lax
from jax.experimental import pallas as pl
from jax.experimental.pallas import tpu as pltpu
from jax.experimental.pallas import tpu_sc as plsc

N = 100000
CIN = 32
COUT = 32
KK = 27
KG = 7
B_, D_, H_, W_ = 4, 64, 64, 64
DP = 66

NP = 102400
CH = 3200
SUB = 128
NSUB = CH // SUB
CHA = 6400
NSUBA = CHA // SUB
NPB = NP // SUB

VOLB = B_ * DP * DP * DP
SAFE = (DP + 1) * DP + 1
VPADR = 4096
VOLP = 1163008
ZROWS = 512
NF = N + ZROWS

MEMW = VOLP // 16
MB = 4096
NMB = MEMW // MB
MREM = MEMW - NMB * MB

_mesh = plsc.VectorSubcoreMesh(core_axis_name="c", subcore_axis_name="s")
_sc_params = pltpu.CompilerParams(use_tc_tiling_on_sc=False)


def _iota16():
    return lax.iota(jnp.int32, 16)


def _sc_gather(cbi, czi, cyi, cxi, fx):
    @functools.partial(
        pl.kernel,
        out_type=(jax.ShapeDtypeStruct((2 * VOLP,), jnp.int32),
                  jax.ShapeDtypeStruct((KG * NPB, SUB, 128), jnp.bfloat16)),
        mesh=_mesh,
        compiler_params=_sc_params,
        scratch_types=[
            pltpu.VMEM((MB,), jnp.int32),
            pltpu.VMEM((CHA,), jnp.int32),
            pltpu.VMEM((CHA,), jnp.int32),
            pltpu.VMEM((CHA,), jnp.int32),
            pltpu.VMEM((CHA,), jnp.int32),
            pltpu.VMEM((NSUBA, SUB), jnp.int32),
            pltpu.VMEM((NSUBA, SUB), jnp.int32),
            pltpu.VMEM((NSUB, SUB), jnp.int32),
            pltpu.VMEM((NSUB, SUB), jnp.int32),
            pltpu.VMEM((NSUB, SUB, CIN), jnp.bfloat16),
            pltpu.SemaphoreType.DMA,
        ],
    )
    def k(cb_h, cz_h, cy_h, cx_h, fx_h, volf, g,
          mbuf, cb, cz, cy, cx, flc, id2, nix, nid, rows, sem):
        c = lax.axis_index("c")
        s = lax.axis_index("s")
        wid = s * 2 + c
        co = c * VOLP

        def fill(i, _):
            mbuf[pl.ds(i * 16, 16)] = N + ((i * 16 + _iota16()) & (ZROWS - 1))
            return 0

        lax.fori_loop(0, MB // 16, fill, 0)

        base_m = co + s * MEMW

        def mset(i, _):
            pltpu.async_copy(mbuf, volf.at[pl.ds(base_m + i * MB, MB)], sem)
            return 0

        lax.fori_loop(0, NMB, mset, 0)
        pltpu.async_copy(mbuf.at[pl.ds(0, MREM)],
                         volf.at[pl.ds(base_m + NMB * MB, MREM)], sem)

        pb = s * CHA
        pltpu.sync_copy(cb_h.at[pl.ds(pb, CHA)], cb)
        pltpu.sync_copy(cz_h.at[pl.ds(pb, CHA)], cz)
        pltpu.sync_copy(cy_h.at[pl.ds(pb, CHA)], cy)
        pltpu.sync_copy(cx_h.at[pl.ds(pb, CHA)], cx)

        def comp(g_, _):
            off = g_ * 16
            r = g_ >> 3
            u = (g_ & 7) * 16
            gid = pb + off + _iota16()
            b16 = cb[pl.ds(off, 16)]
            z16 = cz[pl.ds(off, 16)]
            y16 = cy[pl.ds(off, 16)]
            x16 = cx[pl.ds(off, 16)]
            flat = ((b16 * DP + z16 + 1) * DP + y16 + 1) * DP + x16 + 1
            ispad = gid >= N
            flat = jnp.where(ispad, VOLB + SAFE + (gid & (VPADR - 1)), flat)
            idv = jnp.where(ispad, N + (gid & (ZROWS - 1)), gid)
            flc[r, pl.ds(u, 16)] = co + flat
            id2[r, pl.ds(u, 16)] = idv
            return 0

        lax.fori_loop(0, CHA // 16, comp, 0)

        def mdrain(i, _):
            pltpu.make_async_copy(
                mbuf, volf.at[pl.ds(base_m + i * MB, MB)], sem).wait()
            return 0

        lax.fori_loop(0, NMB, mdrain, 0)
        pltpu.make_async_copy(
            mbuf.at[pl.ds(0, MREM)],
            volf.at[pl.ds(base_m + NMB * MB, MREM)], sem).wait()
        plsc.subcore_barrier()

        def fire(j, _):
            pltpu.async_copy(id2.at[j], volf.at[flc.at[j]], sem)
            return 0

        lax.fori_loop(0, NSUBA, fire, 0)

        def drain(j, _):
            pltpu.make_async_copy(id2.at[j], volf.at[flc.at[j]], sem).wait()
            return 0

        lax.fori_loop(0, NSUBA, drain, 0)
        plsc.subcore_barrier()

        def per_k(kk, _):
            dz = kk // 9 - 1
            dy = (kk // 3) % 3 - 1
            dx = kk % 3 - 1
            delta = jnp.where(kk < KK, (dz * DP + dy) * DP + dx, 0)

            def comp2(g_, _):
                r = g_ >> 3
                u = (g_ & 7) * 16
                nix[r, pl.ds(u, 16)] = flc[c * NSUB + r, pl.ds(u, 16)] + delta
                return 0

            lax.fori_loop(0, CH // 16, comp2, 0)

            def fire1(j, _):
                pltpu.async_copy(volf.at[nix.at[j]], nid.at[j], sem)
                return 0

            lax.fori_loop(0, NSUB, fire1, 0)

            def drain1(j, _):
                pltpu.make_async_copy(volf.at[nix.at[j]], nid.at[j],
                                      sem).wait()
                return 0

            lax.fori_loop(0, NSUB, drain1, 0)

            def fire2(j, _):
                pltpu.async_copy(fx_h.at[nid.at[j]], rows.at[j], sem)
                return 0

            lax.fori_loop(0, NSUB, fire2, 0)

            def drain2(j, _):
                pltpu.make_async_copy(fx_h.at[nid.at[j]], rows.at[j],
                                      sem).wait()
                return 0

            lax.fori_loop(0, NSUB, drain2, 0)

            qg = kk >> 2
            jj = kk & 3
            pltpu.sync_copy(
                rows,
                g.at[pl.ds(qg * NPB + wid * NSUB, NSUB), :,
                     pl.ds(jj * CIN, CIN)])
            return 0

        lax.fori_loop(0, 4 * KG, per_k, 0)

    return k(cbi, czi, cyi, cxi, fx)


def _matmul(g, wf, bias):
    M = 4096

    def body(g_ref, w_ref, b_ref, o_ref):
        k = pl.program_id(1)
        d = jnp.dot(g_ref[...], w_ref[...],
                    preferred_element_type=jnp.float32)

        @pl.when(k == 0)
        def _():
            o_ref[...] = d + b_ref[...]

        @pl.when(k > 0)
        def _():
            o_ref[...] += d

    return pl.pallas_call(
        body,
        grid=(NP // M, KG),
        in_specs=[
            pl.BlockSpec((M, 128), lambda i, k: (k * (NP // M) + i, 0)),
            pl.BlockSpec((128, COUT), lambda i, k: (k, 0)),
            pl.BlockSpec((1, COUT), lambda i, k: (0, 0)),
        ],
        out_specs=pl.BlockSpec((M, COUT), lambda i, k: (i, 0)),
        out_shape=jax.ShapeDtypeStruct((NP, COUT), jnp.float32),
    )(g, wf, bias)


def kernel(feats, coords, W, b):
    dtype_ = feats.dtype
    coords = coords.astype(jnp.int32)
    zpad = jnp.zeros((NP - N,), jnp.int32)
    cbi = jnp.concatenate([coords[:, 0], zpad])
    czi = jnp.concatenate([coords[:, 1], zpad])
    cyi = jnp.concatenate([coords[:, 2], zpad])
    cxi = jnp.concatenate([coords[:, 3], zpad])
    fx = jnp.concatenate(
        [feats.astype(jnp.bfloat16),
         jnp.zeros((ZROWS, CIN), jnp.bfloat16)], axis=0)
    _, g3 = _sc_gather(cbi, czi, cyi, cxi, fx)
    g = g3.reshape(KG * NP, 128)
    wpad = jnp.concatenate(
        [W.astype(jnp.float32),
         jnp.zeros((4 * KG - KK, CIN, COUT), jnp.float32)], axis=0)
    w4 = wpad.reshape(KG * 4 * CIN, COUT).astype(jnp.bfloat16)
    out = _matmul(g, w4, b.reshape(1, COUT).astype(jnp.float32))
    return out[:N].astype(dtype_)


# speedup vs baseline: 4.3667x; 1.0751x over previous
import functools

import jax
import jax.numpy as jnp
from jax import lax
from jax.experimental import pallas as pl
from jax.experimental.pallas import tpu as pltpu
from jax.experimental.pallas import tpu_sc as plsc

N = 100000
CIN = 32
COUT = 32
KK = 27
KG = 7
B_, D_, H_, W_ = 4, 64, 64, 64
DP = 66

NP = 102400
CH = 3200
SUB = 128
NSUB = CH // SUB
CHA = 6400
NSUBA = CHA // SUB
NPB = NP // SUB

VOLB = B_ * DP * DP * DP
SAFE = (DP + 1) * DP + 1
VPADR = 4096
VOLP = 1163008
ZROWS = 512
NF = N + ZROWS

MEMW = VOLP // 16
MB = 8192
NMB = MEMW // MB
MREM = MEMW - NMB * MB

_mesh1 = plsc.VectorSubcoreMesh(core_axis_name="c", subcore_axis_name="s",
                                num_cores=1)
_mesh2 = plsc.VectorSubcoreMesh(core_axis_name="c", subcore_axis_name="s")
_sc_params = pltpu.CompilerParams(use_tc_tiling_on_sc=False)


def _iota16():
    return lax.iota(jnp.int32, 16)


def _build_vol(cbi, czi, cyi, cxi):
    @functools.partial(
        pl.kernel,
        out_type=(jax.ShapeDtypeStruct((VOLP,), jnp.int32),
                  jax.ShapeDtypeStruct((NPB, SUB), jnp.int32)),
        mesh=_mesh1,
        compiler_params=_sc_params,
        scratch_types=[
            pltpu.VMEM((MB,), jnp.int32),
            pltpu.VMEM((CHA,), jnp.int32),
            pltpu.VMEM((CHA,), jnp.int32),
            pltpu.VMEM((CHA,), jnp.int32),
            pltpu.VMEM((CHA,), jnp.int32),
            pltpu.VMEM((NSUBA, SUB), jnp.int32),
            pltpu.VMEM((NSUBA, SUB), jnp.int32),
            pltpu.SemaphoreType.DMA,
        ],
    )
    def k(cb_h, cz_h, cy_h, cx_h, volf, flat3,
          mbuf, cb, cz, cy, cx, fl2, id2, sem):
        s = lax.axis_index("s")

        def fill(i, _):
            mbuf[pl.ds(i * 16, 16)] = N + ((i * 16 + _iota16()) & (ZROWS - 1))
            return 0

        lax.fori_loop(0, MB // 16, fill, 0)

        base_m = s * MEMW

        def mset(i, _):
            pltpu.async_copy(mbuf, volf.at[pl.ds(base_m + i * MB, MB)], sem)
            return 0

        lax.fori_loop(0, NMB, mset, 0)
        pltpu.async_copy(mbuf.at[pl.ds(0, MREM)],
                         volf.at[pl.ds(base_m + NMB * MB, MREM)], sem)

        pb = s * CHA
        pltpu.sync_copy(cb_h.at[pl.ds(pb, CHA)], cb)
        pltpu.sync_copy(cz_h.at[pl.ds(pb, CHA)], cz)
        pltpu.sync_copy(cy_h.at[pl.ds(pb, CHA)], cy)
        pltpu.sync_copy(cx_h.at[pl.ds(pb, CHA)], cx)

        def comp(g_, _):
            off = g_ * 16
            r = g_ >> 3
            u = (g_ & 7) * 16
            gid = pb + off + _iota16()
            b16 = cb[pl.ds(off, 16)]
            z16 = cz[pl.ds(off, 16)]
            y16 = cy[pl.ds(off, 16)]
            x16 = cx[pl.ds(off, 16)]
            flat = ((b16 * DP + z16 + 1) * DP + y16 + 1) * DP + x16 + 1
            ispad = gid >= N
            flat = jnp.where(ispad, VOLB + SAFE + (gid & (VPADR - 1)), flat)
            idv = jnp.where(ispad, N + (gid & (ZROWS - 1)), gid)
            fl2[r, pl.ds(u, 16)] = flat
            id2[r, pl.ds(u, 16)] = idv
            return 0

        lax.fori_loop(0, CHA // 16, comp, 0)
        pltpu.sync_copy(fl2, flat3.at[pl.ds(s * NSUBA, NSUBA), :])

        def mdrain(i, _):
            pltpu.make_async_copy(
                mbuf, volf.at[pl.ds(base_m + i * MB, MB)], sem).wait()
            return 0

        lax.fori_loop(0, NMB, mdrain, 0)
        pltpu.make_async_copy(
            mbuf.at[pl.ds(0, MREM)],
            volf.at[pl.ds(base_m + NMB * MB, MREM)], sem).wait()
        plsc.subcore_barrier()

        def fire(j, _):
            pltpu.async_copy(id2.at[j], volf.at[fl2.at[j]], sem)
            return 0

        lax.fori_loop(0, NSUBA, fire, 0)

        def drain(j, _):
            pltpu.make_async_copy(id2.at[j], volf.at[fl2.at[j]], sem).wait()
            return 0

        lax.fori_loop(0, NSUBA, drain, 0)

    return k(cbi, czi, cyi, cxi)


def _rulebook_gather(volf, flat3, fx):
    @functools.partial(
        pl.kernel,
        out_type=jax.ShapeDtypeStruct((KG * NPB, SUB, 128), jnp.bfloat16),
        mesh=_mesh2,
        compiler_params=_sc_params,
        scratch_types=[
            pltpu.VMEM((NSUB, SUB), jnp.int32),
            pltpu.VMEM((NSUB, SUB), jnp.int32),
            pltpu.VMEM((NSUB, SUB), jnp.int32),
            pltpu.VMEM((NSUB, SUB, CIN), jnp.bfloat16),
            pltpu.SemaphoreType.DMA,
        ],
    )
    def k(volf_h, flat3_h, fx_h, g, fl2, nix, nid, rows, sem):
        c = lax.axis_index("c")
        s = lax.axis_index("s")
        wid = s * 2 + c

        pltpu.sync_copy(flat3_h.at[pl.ds(wid * NSUB, NSUB), :], fl2)

        def per_k(kk, _):
            dz = kk // 9 - 1
            dy = (kk // 3) % 3 - 1
            dx = kk % 3 - 1
            delta = jnp.where(kk < KK, (dz * DP + dy) * DP + dx, 0)

            def comp2(g_, _):
                r = g_ >> 3
                u = (g_ & 7) * 16
                nix[r, pl.ds(u, 16)] = fl2[r, pl.ds(u, 16)] + delta
                return 0

            lax.fori_loop(0, CH // 16, comp2, 0)

            def fire1(j, _):
                pltpu.async_copy(volf_h.at[nix.at[j]], nid.at[j], sem)
                return 0

            lax.fori_loop(0, NSUB, fire1, 0)

            def drain1(j, _):
                pltpu.make_async_copy(volf_h.at[nix.at[j]], nid.at[j],
                                      sem).wait()
                return 0

            lax.fori_loop(0, NSUB, drain1, 0)

            def fire2(j, _):
                pltpu.async_copy(fx_h.at[nid.at[j]], rows.at[j], sem)
                return 0

            lax.fori_loop(0, NSUB, fire2, 0)

            def drain2(j, _):
                pltpu.make_async_copy(fx_h.at[nid.at[j]], rows.at[j],
                                      sem).wait()
                return 0

            lax.fori_loop(0, NSUB, drain2, 0)

            qg = kk >> 2
            jj = kk & 3
            pltpu.sync_copy(
                rows,
                g.at[pl.ds(qg * NPB + wid * NSUB, NSUB), :,
                     pl.ds(jj * CIN, CIN)])
            return 0

        lax.fori_loop(0, 4 * KG, per_k, 0)

    return k(volf, flat3, fx)


def _matmul(g, wf, bias):
    M = 4096

    def body(g_ref, w_ref, b_ref, o_ref):
        k = pl.program_id(1)
        d = jnp.dot(g_ref[...], w_ref[...],
                    preferred_element_type=jnp.float32)

        @pl.when(k == 0)
        def _():
            o_ref[...] = d + b_ref[...]

        @pl.when(k > 0)
        def _():
            o_ref[...] += d

    return pl.pallas_call(
        body,
        grid=(NP // M, KG),
        in_specs=[
            pl.BlockSpec((M, 128), lambda i, k: (k * (NP // M) + i, 0)),
            pl.BlockSpec((128, COUT), lambda i, k: (k, 0)),
            pl.BlockSpec((1, COUT), lambda i, k: (0, 0)),
        ],
        out_specs=pl.BlockSpec((M, COUT), lambda i, k: (i, 0)),
        out_shape=jax.ShapeDtypeStruct((NP, COUT), jnp.float32),
    )(g, wf, bias)


def kernel(feats, coords, W, b):
    dtype_ = feats.dtype
    coords = coords.astype(jnp.int32)
    zpad = jnp.zeros((NP - N,), jnp.int32)
    cbi = jnp.concatenate([coords[:, 0], zpad])
    czi = jnp.concatenate([coords[:, 1], zpad])
    cyi = jnp.concatenate([coords[:, 2], zpad])
    cxi = jnp.concatenate([coords[:, 3], zpad])
    fx = jnp.concatenate(
        [feats.astype(jnp.bfloat16),
         jnp.zeros((ZROWS, CIN), jnp.bfloat16)], axis=0)
    volf, flat3 = _build_vol(cbi, czi, cyi, cxi)
    g3 = _rulebook_gather(volf, flat3, fx)
    g = g3.reshape(KG * NP, 128)
    wpad = jnp.concatenate(
        [W.astype(jnp.float32),
         jnp.zeros((4 * KG - KK, CIN, COUT), jnp.float32)], axis=0)
    w4 = wpad.reshape(KG * 4 * CIN, COUT).astype(jnp.bfloat16)
    out = _matmul(g, w4, b.reshape(1, COUT).astype(jnp.float32))
    return out[:N].astype(dtype_)


# speedup vs baseline: 4.3744x; 1.0018x over previous
import functools

import jax
import jax.numpy as jnp
from jax import lax
from jax.experimental import pallas as pl
from jax.experimental.pallas import tpu as pltpu
from jax.experimental.pallas import tpu_sc as plsc

N = 100000
CIN = 32
COUT = 32
KK = 27
KG = 7
B_, D_, H_, W_ = 4, 64, 64, 64
DP = 66

NP = 102400
CH = 3200
SUB = 128
NSUB = CH // SUB
CHA = 6400
NSUBA = CHA // SUB
NPB = NP // SUB

VOLB = B_ * DP * DP * DP
SAFE = (DP + 1) * DP + 1
VPADR = 4096
VOLP = 1163008
ZROWS = 512
NF = N + ZROWS

MEMW = VOLP // 16
MB = 8192
NMB = MEMW // MB
MREM = MEMW - NMB * MB

_mesh1 = plsc.VectorSubcoreMesh(core_axis_name="c", subcore_axis_name="s",
                                num_cores=1)
_mesh2 = plsc.VectorSubcoreMesh(core_axis_name="c", subcore_axis_name="s")
_sc_params = pltpu.CompilerParams(use_tc_tiling_on_sc=False)


def _iota16():
    return lax.iota(jnp.int32, 16)


def _build_vol(cbi, czi, cyi, cxi):
    @functools.partial(
        pl.kernel,
        out_type=(jax.ShapeDtypeStruct((VOLP,), jnp.int32),
                  jax.ShapeDtypeStruct((NPB, SUB), jnp.int32)),
        mesh=_mesh1,
        compiler_params=_sc_params,
        scratch_types=[
            pltpu.VMEM((MB,), jnp.int32),
            pltpu.VMEM((CHA,), jnp.int32),
            pltpu.VMEM((CHA,), jnp.int32),
            pltpu.VMEM((CHA,), jnp.int32),
            pltpu.VMEM((CHA,), jnp.int32),
            pltpu.VMEM((NSUBA, SUB), jnp.int32),
            pltpu.VMEM((NSUBA, SUB), jnp.int32),
            pltpu.SemaphoreType.DMA,
        ],
    )
    def k(cb_h, cz_h, cy_h, cx_h, volf, flat3,
          mbuf, cb, cz, cy, cx, fl2, id2, sem):
        s = lax.axis_index("s")

        def fill(i, _):
            mbuf[pl.ds(i * 16, 16)] = N + ((i * 16 + _iota16()) & (ZROWS - 1))
            return 0

        lax.fori_loop(0, MB // 16, fill, 0)

        base_m = s * MEMW

        def mset(i, _):
            pltpu.async_copy(mbuf, volf.at[pl.ds(base_m + i * MB, MB)], sem)
            return 0

        lax.fori_loop(0, NMB, mset, 0)
        pltpu.async_copy(mbuf.at[pl.ds(0, MREM)],
                         volf.at[pl.ds(base_m + NMB * MB, MREM)], sem)

        pb = s * CHA
        pltpu.sync_copy(cb_h.at[pl.ds(pb, CHA)], cb)
        pltpu.sync_copy(cz_h.at[pl.ds(pb, CHA)], cz)
        pltpu.sync_copy(cy_h.at[pl.ds(pb, CHA)], cy)
        pltpu.sync_copy(cx_h.at[pl.ds(pb, CHA)], cx)

        def comp(g_, _):
            off = g_ * 16
            r = g_ >> 3
            u = (g_ & 7) * 16
            gid = pb + off + _iota16()
            b16 = cb[pl.ds(off, 16)]
            z16 = cz[pl.ds(off, 16)]
            y16 = cy[pl.ds(off, 16)]
            x16 = cx[pl.ds(off, 16)]
            flat = ((b16 * DP + z16 + 1) * DP + y16 + 1) * DP + x16 + 1
            ispad = gid >= N
            flat = jnp.where(ispad, VOLB + SAFE + (gid & (VPADR - 1)), flat)
            idv = jnp.where(ispad, N + (gid & (ZROWS - 1)), gid)
            fl2[r, pl.ds(u, 16)] = flat
            id2[r, pl.ds(u, 16)] = idv
            return 0

        lax.fori_loop(0, CHA // 16, comp, 0)
        pltpu.sync_copy(fl2, flat3.at[pl.ds(s * NSUBA, NSUBA), :])

        def mdrain(i, _):
            pltpu.make_async_copy(
                mbuf, volf.at[pl.ds(base_m + i * MB, MB)], sem).wait()
            return 0

        lax.fori_loop(0, NMB, mdrain, 0)
        pltpu.make_async_copy(
            mbuf.at[pl.ds(0, MREM)],
            volf.at[pl.ds(base_m + NMB * MB, MREM)], sem).wait()
        plsc.subcore_barrier()

        def fire(j, _):
            pltpu.async_copy(id2.at[j], volf.at[fl2.at[j]], sem)
            return 0

        lax.fori_loop(0, NSUBA, fire, 0)

        def drain(j, _):
            pltpu.make_async_copy(id2.at[j], volf.at[fl2.at[j]], sem).wait()
            return 0

        lax.fori_loop(0, NSUBA, drain, 0)

    return k(cbi, czi, cyi, cxi)


def _rulebook_gather(volf, flat3, fx):
    @functools.partial(
        pl.kernel,
        out_type=jax.ShapeDtypeStruct((KG * NPB, SUB, 128), jnp.bfloat16),
        mesh=_mesh2,
        compiler_params=_sc_params,
        scratch_types=[
            pltpu.VMEM((NSUB, SUB), jnp.int32),
            pltpu.VMEM((NSUB, SUB), jnp.int32),
            pltpu.VMEM((NSUB, SUB), jnp.int32),
            pltpu.VMEM((NSUB, SUB, CIN), jnp.bfloat16),
            pltpu.SemaphoreType.DMA,
        ],
    )
    def k(volf_h, flat3_h, fx_h, g, fl2, nix, nid, rows, sem):
        c = lax.axis_index("c")
        s = lax.axis_index("s")
        wid = s * 2 + c

        pltpu.sync_copy(flat3_h.at[pl.ds(wid * NSUB, NSUB), :], fl2)

        def per_k(kk, _):
            dz = kk // 9 - 1
            dy = (kk // 3) % 3 - 1
            dx = kk % 3 - 1
            delta = jnp.where(kk < KK, (dz * DP + dy) * DP + dx, 0)

            def comp2(g_, _):
                r = g_ >> 3
                u = (g_ & 7) * 16
                nix[r, pl.ds(u, 16)] = fl2[r, pl.ds(u, 16)] + delta
                return 0

            lax.fori_loop(0, CH // 16, comp2, 0)

            def fire1(j, _):
                pltpu.async_copy(volf_h.at[nix.at[j]], nid.at[j], sem)
                return 0

            lax.fori_loop(0, NSUB, fire1, 0)

            def drain1(j, _):
                pltpu.make_async_copy(volf_h.at[nix.at[j]], nid.at[j],
                                      sem).wait()
                return 0

            lax.fori_loop(0, NSUB, drain1, 0)

            def fire2(j, _):
                pltpu.async_copy(fx_h.at[nid.at[j]], rows.at[j], sem)
                return 0

            lax.fori_loop(0, NSUB, fire2, 0)

            def drain2(j, _):
                pltpu.make_async_copy(fx_h.at[nid.at[j]], rows.at[j],
                                      sem).wait()
                return 0

            lax.fori_loop(0, NSUB, drain2, 0)

            qg = kk >> 2
            jj = kk & 3
            pltpu.sync_copy(
                rows,
                g.at[pl.ds(qg * NPB + wid * NSUB, NSUB), :,
                     pl.ds(jj * CIN, CIN)])
            return 0

        lax.fori_loop(0, 4 * KG, per_k, 0)

    return k(volf, flat3, fx)


def _matmul(g3, wf, bias):
    M = 4096
    MB3 = M // SUB

    def body(g_ref, w_ref, b_ref, o_ref):
        k = pl.program_id(1)
        d = jnp.dot(g_ref[...].reshape(M, 128), w_ref[...],
                    preferred_element_type=jnp.float32)

        @pl.when(k == 0)
        def _():
            o_ref[...] = d + b_ref[...]

        @pl.when(k > 0)
        def _():
            o_ref[...] += d

    return pl.pallas_call(
        body,
        grid=(NP // M, KG),
        in_specs=[
            pl.BlockSpec((MB3, SUB, 128),
                         lambda i, k: (k * (NP // M) + i, 0, 0)),
            pl.BlockSpec((128, COUT), lambda i, k: (k, 0)),
            pl.BlockSpec((1, COUT), lambda i, k: (0, 0)),
        ],
        out_specs=pl.BlockSpec((M, COUT), lambda i, k: (i, 0)),
        out_shape=jax.ShapeDtypeStruct((NP, COUT), jnp.float32),
    )(g3, wf, bias)


def kernel(feats, coords, W, b):
    dtype_ = feats.dtype
    coords = coords.astype(jnp.int32)
    zpad = jnp.zeros((NP - N,), jnp.int32)
    cbi = jnp.concatenate([coords[:, 0], zpad])
    czi = jnp.concatenate([coords[:, 1], zpad])
    cyi = jnp.concatenate([coords[:, 2], zpad])
    cxi = jnp.concatenate([coords[:, 3], zpad])
    fx = jnp.concatenate(
        [feats.astype(jnp.bfloat16),
         jnp.zeros((ZROWS, CIN), jnp.bfloat16)], axis=0)
    volf, flat3 = _build_vol(cbi, czi, cyi, cxi)
    g3 = _rulebook_gather(volf, flat3, fx)
    wpad = jnp.concatenate(
        [W.astype(jnp.float32),
         jnp.zeros((4 * KG - KK, CIN, COUT), jnp.float32)], axis=0)
    w4 = wpad.reshape(KG * 4 * CIN, COUT).astype(jnp.bfloat16)
    out = _matmul(g3, w4, b.reshape(1, COUT).astype(jnp.float32))
    return out[:N].astype(dtype_)


# speedup vs baseline: 4.6023x; 1.0521x over previous
import functools

import jax
import jax.numpy as jnp
from jax import lax
from jax.experimental import pallas as pl
from jax.experimental.pallas import tpu as pltpu
from jax.experimental.pallas import tpu_sc as plsc

N = 100000
CIN = 32
COUT = 32
KK = 27
KG = 7
B_, D_, H_, W_ = 4, 64, 64, 64
DP = 66

NP = 102400
CH = 3200
SUB = 128
NSUB = CH // SUB
CHA = 6400
NSUBA = CHA // SUB
NPB = NP // SUB

VOLB = B_ * DP * DP * DP
SAFE = (DP + 1) * DP + 1
VPADR = 4096
VOLP = 1163008
ZROWS = 512
NF = N + ZROWS

MEMW = VOLP // 16
MB = 8192
NMB = MEMW // MB
MREM = MEMW - NMB * MB

_mesh1 = plsc.VectorSubcoreMesh(core_axis_name="c", subcore_axis_name="s",
                                num_cores=1)
_mesh2 = plsc.VectorSubcoreMesh(core_axis_name="c", subcore_axis_name="s")
_sc_params = pltpu.CompilerParams(use_tc_tiling_on_sc=False)


def _iota16():
    return lax.iota(jnp.int32, 16)


def _build_vol(cbi, czi, cyi, cxi):
    @functools.partial(
        pl.kernel,
        out_type=(jax.ShapeDtypeStruct((VOLP,), jnp.int32),
                  jax.ShapeDtypeStruct((NPB, SUB), jnp.int32)),
        mesh=_mesh1,
        compiler_params=_sc_params,
        scratch_types=[
            pltpu.VMEM((MB,), jnp.int32),
            pltpu.VMEM((CHA,), jnp.int32),
            pltpu.VMEM((CHA,), jnp.int32),
            pltpu.VMEM((CHA,), jnp.int32),
            pltpu.VMEM((CHA,), jnp.int32),
            pltpu.VMEM((NSUBA, SUB), jnp.int32),
            pltpu.VMEM((NSUBA, SUB), jnp.int32),
            pltpu.SemaphoreType.DMA,
        ],
    )
    def k(cb_h, cz_h, cy_h, cx_h, volf, flat3,
          mbuf, cb, cz, cy, cx, fl2, id2, sem):
        s = lax.axis_index("s")

        def fill(i, _):
            mbuf[pl.ds(i * 16, 16)] = N + ((i * 16 + _iota16()) & (ZROWS - 1))
            return 0

        lax.fori_loop(0, MB // 16, fill, 0)

        base_m = s * MEMW

        def mset(i, _):
            pltpu.async_copy(mbuf, volf.at[pl.ds(base_m + i * MB, MB)], sem)
            return 0

        lax.fori_loop(0, NMB, mset, 0)
        pltpu.async_copy(mbuf.at[pl.ds(0, MREM)],
                         volf.at[pl.ds(base_m + NMB * MB, MREM)], sem)

        pb = s * CHA
        pltpu.sync_copy(cb_h.at[pl.ds(pb, CHA)], cb)
        pltpu.sync_copy(cz_h.at[pl.ds(pb, CHA)], cz)
        pltpu.sync_copy(cy_h.at[pl.ds(pb, CHA)], cy)
        pltpu.sync_copy(cx_h.at[pl.ds(pb, CHA)], cx)

        def comp(g_, _):
            off = g_ * 16
            r = g_ >> 3
            u = (g_ & 7) * 16
            gid = pb + off + _iota16()
            b16 = cb[pl.ds(off, 16)]
            z16 = cz[pl.ds(off, 16)]
            y16 = cy[pl.ds(off, 16)]
            x16 = cx[pl.ds(off, 16)]
            flat = ((b16 * DP + z16 + 1) * DP + y16 + 1) * DP + x16 + 1
            ispad = gid >= N
            flat = jnp.where(ispad, VOLB + SAFE + (gid & (VPADR - 1)), flat)
            idv = jnp.where(ispad, N + (gid & (ZROWS - 1)), gid)
            fl2[r, pl.ds(u, 16)] = flat
            id2[r, pl.ds(u, 16)] = idv
            return 0

        lax.fori_loop(0, CHA // 16, comp, 0)
        pltpu.sync_copy(fl2, flat3.at[pl.ds(s * NSUBA, NSUBA), :])

        def mdrain(i, _):
            pltpu.make_async_copy(
                mbuf, volf.at[pl.ds(base_m + i * MB, MB)], sem).wait()
            return 0

        lax.fori_loop(0, NMB, mdrain, 0)
        pltpu.make_async_copy(
            mbuf.at[pl.ds(0, MREM)],
            volf.at[pl.ds(base_m + NMB * MB, MREM)], sem).wait()
        plsc.subcore_barrier()

        def fire(j, _):
            pltpu.async_copy(id2.at[j], volf.at[fl2.at[j]], sem)
            return 0

        lax.fori_loop(0, NSUBA, fire, 0)

        def drain(j, _):
            pltpu.make_async_copy(id2.at[j], volf.at[fl2.at[j]], sem).wait()
            return 0

        lax.fori_loop(0, NSUBA, drain, 0)

    return k(cbi, czi, cyi, cxi)


def _rulebook_gather(volf, flat3, fx, q):
    @functools.partial(
        pl.kernel,
        out_type=jax.ShapeDtypeStruct((NPB, SUB, 128), jnp.bfloat16),
        mesh=_mesh2,
        compiler_params=_sc_params,
        scratch_types=[
            pltpu.VMEM((NSUB, SUB), jnp.int32),
            pltpu.VMEM((NSUB, SUB), jnp.int32),
            pltpu.VMEM((NSUB, SUB), jnp.int32),
            pltpu.VMEM((NSUB, SUB, CIN), jnp.bfloat16),
            pltpu.SemaphoreType.DMA,
        ],
    )
    def k(volf_h, flat3_h, fx_h, g, fl2, nix, nid, rows, sem):
        c = lax.axis_index("c")
        s = lax.axis_index("s")
        wid = s * 2 + c

        pltpu.sync_copy(flat3_h.at[pl.ds(wid * NSUB, NSUB), :], fl2)

        def per_k(kk, _):
            dz = kk // 9 - 1
            dy = (kk // 3) % 3 - 1
            dx = kk % 3 - 1
            delta = jnp.where(kk < KK, (dz * DP + dy) * DP + dx, 0)

            def comp2(g_, _):
                r = g_ >> 3
                u = (g_ & 7) * 16
                nix[r, pl.ds(u, 16)] = fl2[r, pl.ds(u, 16)] + delta
                return 0

            lax.fori_loop(0, CH // 16, comp2, 0)

            def fire1(j, _):
                pltpu.async_copy(volf_h.at[nix.at[j]], nid.at[j], sem)
                return 0

            lax.fori_loop(0, NSUB, fire1, 0)

            def drain1(j, _):
                pltpu.make_async_copy(volf_h.at[nix.at[j]], nid.at[j],
                                      sem).wait()
                return 0

            lax.fori_loop(0, NSUB, drain1, 0)

            def fire2(j, _):
                pltpu.async_copy(fx_h.at[nid.at[j]], rows.at[j], sem)
                return 0

            lax.fori_loop(0, NSUB, fire2, 0)

            def drain2(j, _):
                pltpu.make_async_copy(fx_h.at[nid.at[j]], rows.at[j],
                                      sem).wait()
                return 0

            lax.fori_loop(0, NSUB, drain2, 0)

            jj = kk & 3
            pltpu.sync_copy(
                rows,
                g.at[pl.ds(wid * NSUB, NSUB), :,
                     pl.ds(jj * CIN, CIN)])
            return 0

        lax.fori_loop(4 * q, 4 * q + 4, per_k, 0)

    return k(volf, flat3, fx)


def _matmul_acc(acc, g3q, wq):
    M = 4096
    MB3 = M // SUB

    def body(a_ref, g_ref, w_ref, o_ref):
        o_ref[...] = a_ref[...] + jnp.dot(
            g_ref[...].reshape(M, 128), w_ref[...],
            preferred_element_type=jnp.float32)

    return pl.pallas_call(
        body,
        grid=(NP // M,),
        in_specs=[
            pl.BlockSpec((M, COUT), lambda i: (i, 0)),
            pl.BlockSpec((MB3, SUB, 128), lambda i: (i, 0, 0)),
            pl.BlockSpec((128, COUT), lambda i: (0, 0)),
        ],
        out_specs=pl.BlockSpec((M, COUT), lambda i: (i, 0)),
        out_shape=jax.ShapeDtypeStruct((NP, COUT), jnp.float32),
        input_output_aliases={0: 0},
    )(acc, g3q, wq)


def kernel(feats, coords, W, b):
    dtype_ = feats.dtype
    coords = coords.astype(jnp.int32)
    zpad = jnp.zeros((NP - N,), jnp.int32)
    cbi = jnp.concatenate([coords[:, 0], zpad])
    czi = jnp.concatenate([coords[:, 1], zpad])
    cyi = jnp.concatenate([coords[:, 2], zpad])
    cxi = jnp.concatenate([coords[:, 3], zpad])
    fx = jnp.concatenate(
        [feats.astype(jnp.bfloat16),
         jnp.zeros((ZROWS, CIN), jnp.bfloat16)], axis=0)
    volf, flat3 = _build_vol(cbi, czi, cyi, cxi)
    wpad = jnp.concatenate(
        [W.astype(jnp.float32),
         jnp.zeros((4 * KG - KK, CIN, COUT), jnp.float32)], axis=0)
    w4 = wpad.reshape(KG, 4 * CIN, COUT).astype(jnp.bfloat16)
    out = jnp.zeros((NP, COUT), jnp.float32) + b.astype(jnp.float32)
    for q in range(KG):
        g3q = _rulebook_gather(volf, flat3, fx, q)
        out = _matmul_acc(out, g3q, w4[q])
    return out[:N].astype(dtype_)
